# baseline (speedup 1.0000x reference)
import functools

import jax
import jax.numpy as jnp
from jax import lax
from jax.experimental import pallas as pl
from jax.experimental.pallas import tpu as pltpu
from jax.experimental.pallas import tpu_sc as plsc

_EMB = 64
_NTAGS = 5
_LANES = 16
_NC = 2
_NS = 16
_NW = _NC * _NS
_L = 16384
_PER_TILE = _L // _NW
_CHUNK = 128
_NCHUNK = _PER_TILE // _CHUNK
_NGRP = _EMB // _LANES


def _body(words_hbm, table_hbm, w_hbm, bias_hbm, out_hbm,
          idx_v, rows_v, acc_v, w_v, bias_v, out_v, shared, sem):
    cid = lax.axis_index("c")
    sid = lax.axis_index("s")
    tid = cid * _NS + sid
    base = tid * _PER_TILE

    pltpu.sync_copy(words_hbm.at[pl.ds(base, _PER_TILE)], idx_v)
    copies = []
    for j in range(_NCHUNK):
        copies.append(pltpu.async_copy(
            table_hbm.at[idx_v.at[pl.ds(j * _CHUNK, _CHUNK)]],
            rows_v.at[pl.ds(j * _CHUNK, _CHUNK)],
            sem))
    for c in copies:
        c.wait()

    z = jnp.zeros((_LANES,), jnp.float32)

    def _acc(i, carry):
        return tuple(carry[g] + rows_v[i, pl.ds(g * _LANES, _LANES)]
                     for g in range(_NGRP))

    acc = lax.fori_loop(0, _PER_TILE, _acc, (z,) * _NGRP)
    for g in range(_NGRP):
        acc_v[pl.ds(g * _LANES, _LANES)] = acc[g]

    pltpu.sync_copy(acc_v, shared.at[sid])
    plsc.subcore_barrier()

    @pl.when(sid == 0)
    def _finalize():
        pltpu.sync_copy(shared, rows_v.at[pl.ds(0, _NS)])
        pltpu.sync_copy(w_hbm, w_v)
        pltpu.sync_copy(bias_hbm, bias_v)
        s = lax.fori_loop(0, _NS, _acc, (z,) * _NGRP)
        outv = bias_v[...] * (1.0 - cid.astype(jnp.float32))
        for g in range(_NGRP):
            for l in range(_LANES):
                outv = outv + s[g][l] * w_v[g * _LANES + l, :]
        out_v[...] = outv
        pltpu.sync_copy(out_v, out_hbm.at[cid])


_sc_cbow = functools.partial(
    pl.kernel,
    mesh=plsc.VectorSubcoreMesh(core_axis_name="c", subcore_axis_name="s"),
    out_type=jax.ShapeDtypeStruct((_NC, _LANES), jnp.float32),
    compiler_params=pltpu.CompilerParams(use_tc_tiling_on_sc=False),
    scratch_types=[
        pltpu.VMEM((_PER_TILE,), jnp.int32),
        pltpu.VMEM((_PER_TILE, _EMB), jnp.float32),
        pltpu.VMEM((_EMB,), jnp.float32),
        pltpu.VMEM((_EMB, _LANES), jnp.float32),
        pltpu.VMEM((_LANES,), jnp.float32),
        pltpu.VMEM((_LANES,), jnp.float32),
        pltpu.VMEM_SHARED((_NS, _EMB), jnp.float32),
        pltpu.SemaphoreType.DMA,
    ],
)(_body)


def kernel(words, emb_weight, lin_weight, lin_bias):
    words = words.astype(jnp.int32)
    wt_pad = jnp.zeros((_EMB, _LANES), jnp.float32).at[:, :_NTAGS].set(
        lin_weight.astype(jnp.float32).T)
    bias_pad = jnp.zeros((_LANES,), jnp.float32).at[:_NTAGS].set(
        lin_bias.astype(jnp.float32))
    res = _sc_cbow(words, emb_weight, wt_pad, bias_pad)
    return (res[0] + res[1])[:_NTAGS].reshape(1, _NTAGS)
